# Initial kernel scaffold; baseline (speedup 1.0000x reference)
import jax
import jax.numpy as jnp
from jax.experimental import pallas as pl


def kernel(x, edge_index, lin0_w, lin0_b, lin1_w, lin1_b, conv_w1, conv_w2):
    raise NotImplementedError("write your pallas kernel here")


# speedup vs baseline: 1.7571x; 1.7571x over previous
import math

import jax
import jax.numpy as jnp
from jax import lax
from jax.experimental import pallas as pl
from jax.experimental.pallas import tpu as pltpu
from jax.experimental.pallas import tpu_sc as plsc

N = 50000
E = 800000
F_IN = 50
H = 128
C_OUT = 121
ALPHA = 0.1
THETA = 0.5

NC = 2
NS = 16
HALF = N // NC
CS = 12544
NCHUNK = 2
STRIPE = CS // NS
EPT = E // NS
BE = 2000
K = 80
NB = BE // K


def _sc_scatter_body(h_hbm, src_hbm, dst_hbm, zeros_hbm, out_hbm,
                     dst_blk, src_blk, sidx, rows, agg, sem):
    core = lax.axis_index("c")
    tid = lax.axis_index("s")
    ebase = tid * EPT
    sc_lo = core * HALF

    for chunk in range(NCHUNK):
        lo = sc_lo + chunk * CS
        valid = min(CS, HALF - chunk * CS)
        hi = lo + valid

        pltpu.sync_copy(zeros_hbm, agg.at[pl.ds(tid * STRIPE, STRIPE)])
        plsc.subcore_barrier()

        def blk_body(b, _, lo=lo, hi=hi):
            eoff = ebase + b * BE
            pltpu.sync_copy(src_hbm.at[pl.ds(eoff, BE)], src_blk)
            pltpu.sync_copy(dst_hbm.at[pl.ds(eoff, BE)], dst_blk)

            def gs(bb, _):
                for j in range(K // 16):
                    d16 = dst_blk[pl.ds(bb * K + j * 16, 16)]
                    m = (d16 >= lo) & (d16 < hi)
                    sidx[pl.ds(j * 16, 16)] = jnp.where(m, d16 - lo, CS)
                pltpu.async_copy(
                    h_hbm.at[src_blk.at[pl.ds(bb * K, K)]], rows,
                    sem).wait()
                pltpu.sync_copy(rows, agg.at[sidx], add=True)
                return 0

            lax.fori_loop(0, NB, gs, 0)
            return 0

        lax.fori_loop(0, EPT // BE, blk_body, 0)
        plsc.subcore_barrier()

        if valid == CS:
            pltpu.sync_copy(
                agg.at[pl.ds(tid * STRIPE, STRIPE)],
                out_hbm.at[pl.ds(lo + tid * STRIPE, STRIPE)])
        else:
            per = (valid // NS) // 8 * 8
            rem = valid - per * NS
            pltpu.sync_copy(
                agg.at[pl.ds(tid * per, per)],
                out_hbm.at[pl.ds(lo + tid * per, per)])

            @pl.when(tid == 0)
            def _copy_rem():
                pltpu.sync_copy(
                    agg.at[pl.ds(NS * per, rem)],
                    out_hbm.at[pl.ds(lo + NS * per, rem)])
        plsc.subcore_barrier()


def _sc_scatter(h, src, dst, zeros_h):
    mesh = plsc.VectorSubcoreMesh(core_axis_name="c", subcore_axis_name="s",
                                  num_cores=NC, num_subcores=NS)
    return pl.kernel(
        _sc_scatter_body,
        out_type=jax.ShapeDtypeStruct((N, H), jnp.float32),
        mesh=mesh,
        scratch_types=[
            pltpu.VMEM((BE,), jnp.int32),
            pltpu.VMEM((BE,), jnp.int32),
            pltpu.VMEM((K,), jnp.int32),
            pltpu.VMEM((K, H), jnp.float32),
            pltpu.VMEM_SHARED((CS + 8, H), jnp.float32),
            pltpu.SemaphoreType.DMA,
        ],
    )(h, src, dst, zeros_h)


R = 1000


def _lin0_body(x_ref, w_ref, b_ref, o_ref):
    o_ref[...] = jnp.maximum(
        jnp.dot(x_ref[...], w_ref[...],
                preferred_element_type=jnp.float32) + b_ref[...], 0.0)


def _lin0(x, w0t, b0):
    return pl.pallas_call(
        _lin0_body,
        grid=(N // R,),
        in_specs=[pl.BlockSpec((R, F_IN), lambda i: (i, 0)),
                  pl.BlockSpec((F_IN, H), lambda i: (0, 0)),
                  pl.BlockSpec((1, H), lambda i: (0, 0))],
        out_specs=pl.BlockSpec((R, H), lambda i: (i, 0)),
        out_shape=jax.ShapeDtypeStruct((N, H), jnp.float32),
    )(x, w0t, b0)


def _combine1_body(agg_ref, h_ref, w_ref, o_ref):
    out = (1.0 - ALPHA) * agg_ref[...] + ALPHA * h_ref[...]
    o_ref[...] = jnp.maximum(
        jnp.dot(out, w_ref[...], preferred_element_type=jnp.float32)
        + h_ref[...], 0.0)


def _combine1(agg, h, wt1):
    return pl.pallas_call(
        _combine1_body,
        grid=(N // R,),
        in_specs=[pl.BlockSpec((R, H), lambda i: (i, 0)),
                  pl.BlockSpec((R, H), lambda i: (i, 0)),
                  pl.BlockSpec((H, H), lambda i: (0, 0))],
        out_specs=pl.BlockSpec((R, H), lambda i: (i, 0)),
        out_shape=jax.ShapeDtypeStruct((N, H), jnp.float32),
    )(agg, h, wt1)


def _combine2_body(agg_ref, h_ref, xc_ref, w_ref, w1_ref, b1_ref, o_ref):
    out = (1.0 - ALPHA) * agg_ref[...] + ALPHA * h_ref[...]
    xc2 = jnp.maximum(
        jnp.dot(out, w_ref[...], preferred_element_type=jnp.float32)
        + xc_ref[...], 0.0)
    o_ref[...] = jnp.dot(xc2, w1_ref[...],
                         preferred_element_type=jnp.float32) + b1_ref[...]


def _combine2(agg, h, xc1, wt2, w1t, b1):
    return pl.pallas_call(
        _combine2_body,
        grid=(N // R,),
        in_specs=[pl.BlockSpec((R, H), lambda i: (i, 0)),
                  pl.BlockSpec((R, H), lambda i: (i, 0)),
                  pl.BlockSpec((R, H), lambda i: (i, 0)),
                  pl.BlockSpec((H, H), lambda i: (0, 0)),
                  pl.BlockSpec((H, C_OUT), lambda i: (0, 0)),
                  pl.BlockSpec((1, C_OUT), lambda i: (0, 0))],
        out_specs=pl.BlockSpec((R, C_OUT), lambda i: (i, 0)),
        out_shape=jax.ShapeDtypeStruct((N, C_OUT), jnp.float32),
    )(agg, h, xc1, wt2, w1t, b1)


def kernel(x, edge_index, lin0_w, lin0_b, lin1_w, lin1_b, conv_w1, conv_w2):
    src = edge_index[0]
    dst = edge_index[1]
    w0t = lin0_w.T
    b0 = lin0_b.reshape(1, H)
    beta1 = math.log(THETA / 1.0 + 1.0)
    beta2 = math.log(THETA / 2.0 + 1.0)
    eye = jnp.eye(H, dtype=jnp.float32)
    wt1 = (1.0 - beta1) * eye + beta1 * conv_w1
    wt2 = (1.0 - beta2) * eye + beta2 * conv_w2
    w1t = lin1_w.T
    b1 = lin1_b.reshape(1, C_OUT)
    zeros_h = jnp.zeros((STRIPE, H), jnp.float32)

    h = _lin0(x, w0t, b0)
    agg1 = _sc_scatter(h, src, dst, zeros_h)
    xc1 = _combine1(agg1, h, wt1)
    agg2 = _sc_scatter(xc1, src, dst, zeros_h)
    return _combine2(agg2, h, xc1, wt2, w1t, b1)


# speedup vs baseline: 2.1715x; 1.2358x over previous
import math

import jax
import jax.numpy as jnp
from jax import lax
from jax.experimental import pallas as pl
from jax.experimental.pallas import tpu as pltpu
from jax.experimental.pallas import tpu_sc as plsc

N = 50000
E = 800000
F_IN = 50
H = 128
C_OUT = 121
ALPHA = 0.1
THETA = 0.5

NC = 2
NS = 16
HALF = N // NC
CS = 12544
NCHUNK = 2
STRIPE = CS // NS
EPT = E // NS
BE = 2000
K = 80
NB = BE // K


def _sc_scatter_body(h_hbm, src_hbm, dst_hbm, zeros_hbm, out_hbm,
                     dst_blk, src_blk, sidx0, sidx1, rows0, rows1,
                     agg, gsem0, gsem1, ssem0, ssem1):
    core = lax.axis_index("c")
    tid = lax.axis_index("s")
    ebase = tid * EPT
    sc_lo = core * HALF

    for chunk in range(NCHUNK):
        lo = sc_lo + chunk * CS
        valid = min(CS, HALF - chunk * CS)
        hi = lo + valid

        pltpu.sync_copy(zeros_hbm, agg.at[pl.ds(tid * STRIPE, STRIPE)])
        plsc.subcore_barrier()

        def blk_body(b, _, lo=lo, hi=hi):
            eoff = ebase + b * BE
            pltpu.sync_copy(src_hbm.at[pl.ds(eoff, BE)], src_blk)
            pltpu.sync_copy(dst_hbm.at[pl.ds(eoff, BE)], dst_blk)

            def sidx_of(bb, sidx):
                for j in range(K // 16):
                    d16 = dst_blk[pl.ds(bb * K + j * 16, 16)]
                    m = (d16 >= lo) & (d16 < hi)
                    sidx[pl.ds(j * 16, 16)] = jnp.where(m, d16 - lo, CS)

            def gather(bb, rows, sem):
                pltpu.async_copy(
                    h_hbm.at[src_blk.at[pl.ds(bb * K, K)]], rows, sem)

            def gwait(rows, sem):
                pltpu.make_async_copy(
                    h_hbm.at[src_blk.at[pl.ds(0, K)]], rows, sem).wait()

            def scat(rows, sidx, sem):
                pltpu.async_copy(rows, agg.at[sidx], sem, add=True)

            def swait(rows, sidx, sem):
                pltpu.make_async_copy(rows, agg.at[sidx], sem).wait()

            sidx_of(0, sidx0)
            gather(0, rows0, gsem0)
            gwait(rows0, gsem0)
            scat(rows0, sidx0, ssem0)
            sidx_of(1, sidx1)
            gather(1, rows1, gsem1)
            gwait(rows1, gsem1)
            scat(rows1, sidx1, ssem1)
            swait(rows0, sidx0, ssem0)
            sidx_of(2, sidx0)
            gather(2, rows0, gsem0)

            def pair(i, _):
                b0 = 2 * i
                gwait(rows0, gsem0)
                scat(rows0, sidx0, ssem0)
                swait(rows1, sidx1, ssem1)
                sidx_of(b0 + 1, sidx1)
                gather(b0 + 1, rows1, gsem1)
                gwait(rows1, gsem1)
                scat(rows1, sidx1, ssem1)
                swait(rows0, sidx0, ssem0)
                sidx_of(b0 + 2, sidx0)
                gather(b0 + 2, rows0, gsem0)
                return 0

            lax.fori_loop(1, (NB - 1) // 2, pair, 0)
            gwait(rows0, gsem0)
            scat(rows0, sidx0, ssem0)
            swait(rows1, sidx1, ssem1)
            swait(rows0, sidx0, ssem0)
            return 0

        lax.fori_loop(0, EPT // BE, blk_body, 0)
        plsc.subcore_barrier()

        if valid == CS:
            pltpu.sync_copy(
                agg.at[pl.ds(tid * STRIPE, STRIPE)],
                out_hbm.at[pl.ds(lo + tid * STRIPE, STRIPE)])
        else:
            per = (valid // NS) // 8 * 8
            rem = valid - per * NS
            pltpu.sync_copy(
                agg.at[pl.ds(tid * per, per)],
                out_hbm.at[pl.ds(lo + tid * per, per)])

            @pl.when(tid == 0)
            def _copy_rem():
                pltpu.sync_copy(
                    agg.at[pl.ds(NS * per, rem)],
                    out_hbm.at[pl.ds(lo + NS * per, rem)])
        plsc.subcore_barrier()


def _sc_scatter(h, src, dst, zeros_h):
    mesh = plsc.VectorSubcoreMesh(core_axis_name="c", subcore_axis_name="s",
                                  num_cores=NC, num_subcores=NS)
    return pl.kernel(
        _sc_scatter_body,
        out_type=jax.ShapeDtypeStruct((N, H), jnp.float32),
        mesh=mesh,
        scratch_types=[
            pltpu.VMEM((BE,), jnp.int32),
            pltpu.VMEM((BE,), jnp.int32),
            pltpu.VMEM((K,), jnp.int32),
            pltpu.VMEM((K,), jnp.int32),
            pltpu.VMEM((K, H), jnp.float32),
            pltpu.VMEM((K, H), jnp.float32),
            pltpu.VMEM_SHARED((CS + 8, H), jnp.float32),
            pltpu.SemaphoreType.DMA,
            pltpu.SemaphoreType.DMA,
            pltpu.SemaphoreType.DMA,
            pltpu.SemaphoreType.DMA,
        ],
    )(h, src, dst, zeros_h)


R = 1000


def _lin0_body(x_ref, w_ref, b_ref, o_ref):
    o_ref[...] = jnp.maximum(
        jnp.dot(x_ref[...], w_ref[...],
                preferred_element_type=jnp.float32) + b_ref[...], 0.0)


def _lin0(x, w0t, b0):
    return pl.pallas_call(
        _lin0_body,
        grid=(N // R,),
        in_specs=[pl.BlockSpec((R, F_IN), lambda i: (i, 0)),
                  pl.BlockSpec((F_IN, H), lambda i: (0, 0)),
                  pl.BlockSpec((1, H), lambda i: (0, 0))],
        out_specs=pl.BlockSpec((R, H), lambda i: (i, 0)),
        out_shape=jax.ShapeDtypeStruct((N, H), jnp.float32),
    )(x, w0t, b0)


def _combine1_body(agg_ref, h_ref, w_ref, o_ref):
    out = (1.0 - ALPHA) * agg_ref[...] + ALPHA * h_ref[...]
    o_ref[...] = jnp.maximum(
        jnp.dot(out, w_ref[...], preferred_element_type=jnp.float32)
        + h_ref[...], 0.0)


def _combine1(agg, h, wt1):
    return pl.pallas_call(
        _combine1_body,
        grid=(N // R,),
        in_specs=[pl.BlockSpec((R, H), lambda i: (i, 0)),
                  pl.BlockSpec((R, H), lambda i: (i, 0)),
                  pl.BlockSpec((H, H), lambda i: (0, 0))],
        out_specs=pl.BlockSpec((R, H), lambda i: (i, 0)),
        out_shape=jax.ShapeDtypeStruct((N, H), jnp.float32),
    )(agg, h, wt1)


def _combine2_body(agg_ref, h_ref, xc_ref, w_ref, w1_ref, b1_ref, o_ref):
    out = (1.0 - ALPHA) * agg_ref[...] + ALPHA * h_ref[...]
    xc2 = jnp.maximum(
        jnp.dot(out, w_ref[...], preferred_element_type=jnp.float32)
        + xc_ref[...], 0.0)
    o_ref[...] = jnp.dot(xc2, w1_ref[...],
                         preferred_element_type=jnp.float32) + b1_ref[...]


def _combine2(agg, h, xc1, wt2, w1t, b1):
    return pl.pallas_call(
        _combine2_body,
        grid=(N // R,),
        in_specs=[pl.BlockSpec((R, H), lambda i: (i, 0)),
                  pl.BlockSpec((R, H), lambda i: (i, 0)),
                  pl.BlockSpec((R, H), lambda i: (i, 0)),
                  pl.BlockSpec((H, H), lambda i: (0, 0)),
                  pl.BlockSpec((H, C_OUT), lambda i: (0, 0)),
                  pl.BlockSpec((1, C_OUT), lambda i: (0, 0))],
        out_specs=pl.BlockSpec((R, C_OUT), lambda i: (i, 0)),
        out_shape=jax.ShapeDtypeStruct((N, C_OUT), jnp.float32),
    )(agg, h, xc1, wt2, w1t, b1)


def kernel(x, edge_index, lin0_w, lin0_b, lin1_w, lin1_b, conv_w1, conv_w2):
    src = edge_index[0]
    dst = edge_index[1]
    w0t = lin0_w.T
    b0 = lin0_b.reshape(1, H)
    beta1 = math.log(THETA / 1.0 + 1.0)
    beta2 = math.log(THETA / 2.0 + 1.0)
    eye = jnp.eye(H, dtype=jnp.float32)
    wt1 = (1.0 - beta1) * eye + beta1 * conv_w1
    wt2 = (1.0 - beta2) * eye + beta2 * conv_w2
    w1t = lin1_w.T
    b1 = lin1_b.reshape(1, C_OUT)
    zeros_h = jnp.zeros((STRIPE, H), jnp.float32)

    h = _lin0(x, w0t, b0)
    agg1 = _sc_scatter(h, src, dst, zeros_h)
    xc1 = _combine1(agg1, h, wt1)
    agg2 = _sc_scatter(xc1, src, dst, zeros_h)
    return _combine2(agg2, h, xc1, wt2, w1t, b1)


# speedup vs baseline: 3.7954x; 1.7478x over previous
import math

import jax
import jax.numpy as jnp
from jax import lax
from jax.experimental import pallas as pl
from jax.experimental.pallas import tpu as pltpu
from jax.experimental.pallas import tpu_sc as plsc

N = 50000
E = 800000
F_IN = 50
H = 128
C_OUT = 121
ALPHA = 0.1
THETA = 0.5

NC = 2
NS = 16
G = 4
GC = H // G
K = 125
EK = E // K
IRT = EK // NS
BRK = 40
NBLK = IRT // BRK
ZST = 3200
ZREM = N - 15 * ZST


def _sc_pass(table, out_g, src2_hbm, dst2_hbm, zeros_hbm,
             dst_blk, src_blk, rows0, rows1, agg,
             gsem0, gsem1, ssem0, ssem1, tid):
    @pl.when(tid < 15)
    def _zero_main():
        pltpu.sync_copy(zeros_hbm, agg.at[pl.ds(tid * ZST, ZST)])

    @pl.when(tid == 15)
    def _zero_rem():
        pltpu.sync_copy(zeros_hbm.at[pl.ds(0, ZREM)],
                        agg.at[pl.ds(15 * ZST, ZREM)])

    plsc.subcore_barrier()

    def blk_body(b, _):
        row_off = tid * IRT + b * BRK
        pltpu.sync_copy(src2_hbm.at[pl.ds(row_off, BRK)], src_blk)
        pltpu.sync_copy(dst2_hbm.at[pl.ds(row_off, BRK)], dst_blk)

        def gather(bb, rows, sem):
            pltpu.async_copy(table.at[src_blk.at[bb]], rows, sem)

        def gwait(rows, sem):
            pltpu.make_async_copy(table.at[src_blk.at[0]], rows, sem).wait()

        def scat(bb, rows, sem):
            pltpu.async_copy(rows, agg.at[dst_blk.at[bb]], sem, add=True)

        def swait(rows, sem):
            pltpu.make_async_copy(rows, agg.at[dst_blk.at[0]], sem).wait()

        gather(0, rows0, gsem0)
        gwait(rows0, gsem0)
        scat(0, rows0, ssem0)
        gather(1, rows1, gsem1)
        gwait(rows1, gsem1)
        scat(1, rows1, ssem1)
        swait(rows0, ssem0)
        gather(2, rows0, gsem0)

        def pair(i, _):
            b0 = 2 * i
            gwait(rows0, gsem0)
            scat(b0, rows0, ssem0)
            swait(rows1, ssem1)
            gather(b0 + 1, rows1, gsem1)
            gwait(rows1, gsem1)
            scat(b0 + 1, rows1, ssem1)
            swait(rows0, ssem0)
            gather(b0 + 2, rows0, gsem0)
            return 0

        lax.fori_loop(1, (BRK - 2) // 2, pair, 0)
        gwait(rows0, gsem0)
        scat(BRK - 2, rows0, ssem0)
        swait(rows1, ssem1)
        gather(BRK - 1, rows1, gsem1)
        gwait(rows1, gsem1)
        scat(BRK - 1, rows1, ssem1)
        swait(rows0, ssem0)
        swait(rows1, ssem1)
        return 0

    lax.fori_loop(0, NBLK, blk_body, 0)
    plsc.subcore_barrier()

    @pl.when(tid < 15)
    def _out_main():
        pltpu.sync_copy(agg.at[pl.ds(tid * ZST, ZST)],
                        out_g.at[pl.ds(tid * ZST, ZST)])

    @pl.when(tid == 15)
    def _out_rem():
        pltpu.sync_copy(agg.at[pl.ds(15 * ZST, ZREM)],
                        out_g.at[pl.ds(15 * ZST, ZREM)])

    plsc.subcore_barrier()


def _sc_scatter_body(h4_hbm, src2_hbm, dst2_hbm, zeros_hbm, out_hbm,
                     dst_blk, src_blk, rows0, rows1, agg,
                     gsem0, gsem1, ssem0, ssem1):
    core = lax.axis_index("c")
    tid = lax.axis_index("s")

    for c in range(NC):
        @pl.when(core == c)
        def _core_work(c=c):
            for gi in range(G // NC):
                g = c * (G // NC) + gi
                _sc_pass(h4_hbm.at[g], out_hbm.at[g], src2_hbm, dst2_hbm,
                         zeros_hbm, dst_blk, src_blk, rows0, rows1, agg,
                         gsem0, gsem1, ssem0, ssem1, tid)


def _sc_scatter(h4, src2, dst2, zeros32):
    mesh = plsc.VectorSubcoreMesh(core_axis_name="c", subcore_axis_name="s",
                                  num_cores=NC, num_subcores=NS)
    return pl.kernel(
        _sc_scatter_body,
        out_type=jax.ShapeDtypeStruct((G, N, GC), jnp.float32),
        mesh=mesh,
        scratch_types=[
            pltpu.VMEM((BRK, K), jnp.int32),
            pltpu.VMEM((BRK, K), jnp.int32),
            pltpu.VMEM((K, GC), jnp.float32),
            pltpu.VMEM((K, GC), jnp.float32),
            pltpu.VMEM_SHARED((N, GC), jnp.float32),
            pltpu.SemaphoreType.DMA,
            pltpu.SemaphoreType.DMA,
            pltpu.SemaphoreType.DMA,
            pltpu.SemaphoreType.DMA,
        ],
        compiler_params=pltpu.CompilerParams(use_tc_tiling_on_sc=False),
    )(h4, src2, dst2, zeros32)


R = 1000


def _split4(x):
    return [x[:, c * GC:(c + 1) * GC] for c in range(G)]


def _lin0_body(x_ref, w_ref, b_ref, o_ref, o4_ref):
    h = jnp.maximum(
        jnp.dot(x_ref[...], w_ref[...],
                preferred_element_type=jnp.float32) + b_ref[...], 0.0)
    o_ref[...] = h
    for c in range(G):
        o4_ref[c] = h[:, c * GC:(c + 1) * GC]


def _lin0(x, w0t, b0):
    return pl.pallas_call(
        _lin0_body,
        grid=(N // R,),
        in_specs=[pl.BlockSpec((R, F_IN), lambda i: (i, 0)),
                  pl.BlockSpec((F_IN, H), lambda i: (0, 0)),
                  pl.BlockSpec((1, H), lambda i: (0, 0))],
        out_specs=[pl.BlockSpec((R, H), lambda i: (i, 0)),
                   pl.BlockSpec((G, R, GC), lambda i: (0, i, 0))],
        out_shape=[jax.ShapeDtypeStruct((N, H), jnp.float32),
                   jax.ShapeDtypeStruct((G, N, GC), jnp.float32)],
    )(x, w0t, b0)


def _combine1_body(a4_ref, h_ref, w_ref, o_ref, o4_ref):
    a4 = a4_ref[...]
    agg = jnp.concatenate([a4[c] for c in range(G)], axis=-1)
    out = (1.0 - ALPHA) * agg + ALPHA * h_ref[...]
    xc = jnp.maximum(
        jnp.dot(out, w_ref[...], preferred_element_type=jnp.float32)
        + h_ref[...], 0.0)
    o_ref[...] = xc
    for c in range(G):
        o4_ref[c] = xc[:, c * GC:(c + 1) * GC]


def _combine1(agg4, h, wt1):
    return pl.pallas_call(
        _combine1_body,
        grid=(N // R,),
        in_specs=[pl.BlockSpec((G, R, GC), lambda i: (0, i, 0)),
                  pl.BlockSpec((R, H), lambda i: (i, 0)),
                  pl.BlockSpec((H, H), lambda i: (0, 0))],
        out_specs=[pl.BlockSpec((R, H), lambda i: (i, 0)),
                   pl.BlockSpec((G, R, GC), lambda i: (0, i, 0))],
        out_shape=[jax.ShapeDtypeStruct((N, H), jnp.float32),
                   jax.ShapeDtypeStruct((G, N, GC), jnp.float32)],
    )(agg4, h, wt1)


def _combine2_body(a4_ref, h_ref, xc_ref, w_ref, w1_ref, b1_ref, o_ref):
    a4 = a4_ref[...]
    agg = jnp.concatenate([a4[c] for c in range(G)], axis=-1)
    out = (1.0 - ALPHA) * agg + ALPHA * h_ref[...]
    xc2 = jnp.maximum(
        jnp.dot(out, w_ref[...], preferred_element_type=jnp.float32)
        + xc_ref[...], 0.0)
    o_ref[...] = jnp.dot(xc2, w1_ref[...],
                         preferred_element_type=jnp.float32) + b1_ref[...]


def _combine2(agg4, h, xc1, wt2, w1t, b1):
    return pl.pallas_call(
        _combine2_body,
        grid=(N // R,),
        in_specs=[pl.BlockSpec((G, R, GC), lambda i: (0, i, 0)),
                  pl.BlockSpec((R, H), lambda i: (i, 0)),
                  pl.BlockSpec((R, H), lambda i: (i, 0)),
                  pl.BlockSpec((H, H), lambda i: (0, 0)),
                  pl.BlockSpec((H, C_OUT), lambda i: (0, 0)),
                  pl.BlockSpec((1, C_OUT), lambda i: (0, 0))],
        out_specs=pl.BlockSpec((R, C_OUT), lambda i: (i, 0)),
        out_shape=jax.ShapeDtypeStruct((N, C_OUT), jnp.float32),
    )(agg4, h, xc1, wt2, w1t, b1)


def kernel(x, edge_index, lin0_w, lin0_b, lin1_w, lin1_b, conv_w1, conv_w2):
    src2 = edge_index[0].reshape(EK, K)
    dst2 = edge_index[1].reshape(EK, K)
    w0t = lin0_w.T
    b0 = lin0_b.reshape(1, H)
    beta1 = math.log(THETA / 1.0 + 1.0)
    beta2 = math.log(THETA / 2.0 + 1.0)
    eye = jnp.eye(H, dtype=jnp.float32)
    wt1 = (1.0 - beta1) * eye + beta1 * conv_w1
    wt2 = (1.0 - beta2) * eye + beta2 * conv_w2
    w1t = lin1_w.T
    b1 = lin1_b.reshape(1, C_OUT)
    zeros32 = jnp.zeros((ZST, GC), jnp.float32)

    h, h4 = _lin0(x, w0t, b0)
    agg4 = _sc_scatter(h4, src2, dst2, zeros32)
    xc1, xc14 = _combine1(agg4, h, wt1)
    agg4b = _sc_scatter(xc14, src2, dst2, zeros32)
    return _combine2(agg4b, h, xc1, wt2, w1t, b1)


# speedup vs baseline: 4.6882x; 1.2353x over previous
import math

import jax
import jax.numpy as jnp
from jax import lax
from jax.experimental import pallas as pl
from jax.experimental.pallas import tpu as pltpu
from jax.experimental.pallas import tpu_sc as plsc

N = 50000
E = 800000
F_IN = 50
H = 128
C_OUT = 121
ALPHA = 0.1
THETA = 0.5

NC = 2
NS = 16
G = 4
GC = H // G
K = 250
EK = E // K
IRT = EK // NS
BRK = 8
NBLK = IRT // BRK
ZST = 3200
ZREM = N - 15 * ZST


def _sc_pass(table, out_g, src2_hbm, dst2_hbm, zeros_hbm,
             dst_blk, src_blk, rows0, rows1, agg,
             gsem0, gsem1, ssem0, ssem1, tid):
    @pl.when(tid < 15)
    def _zero_main():
        pltpu.sync_copy(zeros_hbm, agg.at[pl.ds(tid * ZST, ZST)])

    @pl.when(tid == 15)
    def _zero_rem():
        pltpu.sync_copy(zeros_hbm.at[pl.ds(0, ZREM)],
                        agg.at[pl.ds(15 * ZST, ZREM)])

    plsc.subcore_barrier()

    def blk_body(b, _):
        row_off = tid * IRT + b * BRK
        pltpu.sync_copy(src2_hbm.at[pl.ds(row_off, BRK)], src_blk)
        pltpu.sync_copy(dst2_hbm.at[pl.ds(row_off, BRK)], dst_blk)

        def gather(bb, rows, sem):
            pltpu.async_copy(table.at[src_blk.at[bb]], rows, sem)

        def gwait(rows, sem):
            pltpu.make_async_copy(table.at[src_blk.at[0]], rows, sem).wait()

        def scat(bb, rows, sem):
            pltpu.async_copy(rows, agg.at[dst_blk.at[bb]], sem, add=True)

        def swait(rows, sem):
            pltpu.make_async_copy(rows, agg.at[dst_blk.at[0]], sem).wait()

        gather(0, rows0, gsem0)
        gwait(rows0, gsem0)
        scat(0, rows0, ssem0)
        gather(1, rows1, gsem1)
        gwait(rows1, gsem1)
        scat(1, rows1, ssem1)
        swait(rows0, ssem0)
        gather(2, rows0, gsem0)

        def pair(i, _):
            b0 = 2 * i
            gwait(rows0, gsem0)
            scat(b0, rows0, ssem0)
            swait(rows1, ssem1)
            gather(b0 + 1, rows1, gsem1)
            gwait(rows1, gsem1)
            scat(b0 + 1, rows1, ssem1)
            swait(rows0, ssem0)
            gather(b0 + 2, rows0, gsem0)
            return 0

        lax.fori_loop(1, (BRK - 2) // 2, pair, 0)
        gwait(rows0, gsem0)
        scat(BRK - 2, rows0, ssem0)
        swait(rows1, ssem1)
        gather(BRK - 1, rows1, gsem1)
        gwait(rows1, gsem1)
        scat(BRK - 1, rows1, ssem1)
        swait(rows0, ssem0)
        swait(rows1, ssem1)
        return 0

    lax.fori_loop(0, NBLK, blk_body, 0)
    plsc.subcore_barrier()

    @pl.when(tid < 15)
    def _out_main():
        pltpu.sync_copy(agg.at[pl.ds(tid * ZST, ZST)],
                        out_g.at[pl.ds(tid * ZST, ZST)])

    @pl.when(tid == 15)
    def _out_rem():
        pltpu.sync_copy(agg.at[pl.ds(15 * ZST, ZREM)],
                        out_g.at[pl.ds(15 * ZST, ZREM)])

    plsc.subcore_barrier()


def _sc_scatter_body(h4_hbm, src2_hbm, dst2_hbm, zeros_hbm, out_hbm,
                     dst_blk, src_blk, rows0, rows1, agg,
                     gsem0, gsem1, ssem0, ssem1):
    core = lax.axis_index("c")
    tid = lax.axis_index("s")

    for c in range(NC):
        @pl.when(core == c)
        def _core_work(c=c):
            for gi in range(G // NC):
                g = c * (G // NC) + gi
                _sc_pass(h4_hbm.at[g], out_hbm.at[g], src2_hbm, dst2_hbm,
                         zeros_hbm, dst_blk, src_blk, rows0, rows1, agg,
                         gsem0, gsem1, ssem0, ssem1, tid)


def _sc_scatter(h4, src2, dst2, zeros32):
    mesh = plsc.VectorSubcoreMesh(core_axis_name="c", subcore_axis_name="s",
                                  num_cores=NC, num_subcores=NS)
    return pl.kernel(
        _sc_scatter_body,
        out_type=jax.ShapeDtypeStruct((G, N, GC), jnp.float32),
        mesh=mesh,
        scratch_types=[
            pltpu.VMEM((BRK, K), jnp.int32),
            pltpu.VMEM((BRK, K), jnp.int32),
            pltpu.VMEM((K, GC), jnp.float32),
            pltpu.VMEM((K, GC), jnp.float32),
            pltpu.VMEM_SHARED((N, GC), jnp.float32),
            pltpu.SemaphoreType.DMA,
            pltpu.SemaphoreType.DMA,
            pltpu.SemaphoreType.DMA,
            pltpu.SemaphoreType.DMA,
        ],
        compiler_params=pltpu.CompilerParams(use_tc_tiling_on_sc=False),
    )(h4, src2, dst2, zeros32)


R = 1000


def _split4(x):
    return [x[:, c * GC:(c + 1) * GC] for c in range(G)]


def _lin0_body(x_ref, w_ref, b_ref, o_ref, o4_ref):
    h = jnp.maximum(
        jnp.dot(x_ref[...], w_ref[...],
                preferred_element_type=jnp.float32) + b_ref[...], 0.0)
    o_ref[...] = h
    for c in range(G):
        o4_ref[c] = h[:, c * GC:(c + 1) * GC]


def _lin0(x, w0t, b0):
    return pl.pallas_call(
        _lin0_body,
        grid=(N // R,),
        in_specs=[pl.BlockSpec((R, F_IN), lambda i: (i, 0)),
                  pl.BlockSpec((F_IN, H), lambda i: (0, 0)),
                  pl.BlockSpec((1, H), lambda i: (0, 0))],
        out_specs=[pl.BlockSpec((R, H), lambda i: (i, 0)),
                   pl.BlockSpec((G, R, GC), lambda i: (0, i, 0))],
        out_shape=[jax.ShapeDtypeStruct((N, H), jnp.float32),
                   jax.ShapeDtypeStruct((G, N, GC), jnp.float32)],
    )(x, w0t, b0)


def _combine1_body(a4_ref, h_ref, w_ref, o_ref, o4_ref):
    a4 = a4_ref[...]
    agg = jnp.concatenate([a4[c] for c in range(G)], axis=-1)
    out = (1.0 - ALPHA) * agg + ALPHA * h_ref[...]
    xc = jnp.maximum(
        jnp.dot(out, w_ref[...], preferred_element_type=jnp.float32)
        + h_ref[...], 0.0)
    o_ref[...] = xc
    for c in range(G):
        o4_ref[c] = xc[:, c * GC:(c + 1) * GC]


def _combine1(agg4, h, wt1):
    return pl.pallas_call(
        _combine1_body,
        grid=(N // R,),
        in_specs=[pl.BlockSpec((G, R, GC), lambda i: (0, i, 0)),
                  pl.BlockSpec((R, H), lambda i: (i, 0)),
                  pl.BlockSpec((H, H), lambda i: (0, 0))],
        out_specs=[pl.BlockSpec((R, H), lambda i: (i, 0)),
                   pl.BlockSpec((G, R, GC), lambda i: (0, i, 0))],
        out_shape=[jax.ShapeDtypeStruct((N, H), jnp.float32),
                   jax.ShapeDtypeStruct((G, N, GC), jnp.float32)],
    )(agg4, h, wt1)


def _combine2_body(a4_ref, h_ref, xc_ref, w_ref, w1_ref, b1_ref, o_ref):
    a4 = a4_ref[...]
    agg = jnp.concatenate([a4[c] for c in range(G)], axis=-1)
    out = (1.0 - ALPHA) * agg + ALPHA * h_ref[...]
    xc2 = jnp.maximum(
        jnp.dot(out, w_ref[...], preferred_element_type=jnp.float32)
        + xc_ref[...], 0.0)
    o_ref[...] = jnp.dot(xc2, w1_ref[...],
                         preferred_element_type=jnp.float32) + b1_ref[...]


def _combine2(agg4, h, xc1, wt2, w1t, b1):
    return pl.pallas_call(
        _combine2_body,
        grid=(N // R,),
        in_specs=[pl.BlockSpec((G, R, GC), lambda i: (0, i, 0)),
                  pl.BlockSpec((R, H), lambda i: (i, 0)),
                  pl.BlockSpec((R, H), lambda i: (i, 0)),
                  pl.BlockSpec((H, H), lambda i: (0, 0)),
                  pl.BlockSpec((H, C_OUT), lambda i: (0, 0)),
                  pl.BlockSpec((1, C_OUT), lambda i: (0, 0))],
        out_specs=pl.BlockSpec((R, C_OUT), lambda i: (i, 0)),
        out_shape=jax.ShapeDtypeStruct((N, C_OUT), jnp.float32),
    )(agg4, h, xc1, wt2, w1t, b1)


def kernel(x, edge_index, lin0_w, lin0_b, lin1_w, lin1_b, conv_w1, conv_w2):
    src2 = edge_index[0].reshape(EK, K)
    dst2 = edge_index[1].reshape(EK, K)
    w0t = lin0_w.T
    b0 = lin0_b.reshape(1, H)
    beta1 = math.log(THETA / 1.0 + 1.0)
    beta2 = math.log(THETA / 2.0 + 1.0)
    eye = jnp.eye(H, dtype=jnp.float32)
    wt1 = (1.0 - beta1) * eye + beta1 * conv_w1
    wt2 = (1.0 - beta2) * eye + beta2 * conv_w2
    w1t = lin1_w.T
    b1 = lin1_b.reshape(1, C_OUT)
    zeros32 = jnp.zeros((ZST, GC), jnp.float32)

    h, h4 = _lin0(x, w0t, b0)
    agg4 = _sc_scatter(h4, src2, dst2, zeros32)
    xc1, xc14 = _combine1(agg4, h, wt1)
    agg4b = _sc_scatter(xc14, src2, dst2, zeros32)
    return _combine2(agg4b, h, xc1, wt2, w1t, b1)
